# baseline (speedup 1.0000x reference)
import jax
import jax.numpy as jnp
from jax.experimental import pallas as pl
from jax.experimental.pallas import tpu as pltpu

_BB = 8


def _add_body(in_ref, tab_ref, out_ref):
    out_ref[...] = in_ref[...] + tab_ref[...]


def kernel(inputs, pos_table):
    B, T, D = inputs.shape
    grid = (B // _BB,)
    return pl.pallas_call(
        _add_body,
        grid=grid,
        in_specs=[
            pl.BlockSpec((_BB, T, D), lambda i: (i, 0, 0)),
            pl.BlockSpec((1, T, D), lambda i: (0, 0, 0)),
        ],
        out_specs=pl.BlockSpec((_BB, T, D), lambda i: (i, 0, 0)),
        out_shape=jax.ShapeDtypeStruct((B, T, D), inputs.dtype),
    )(inputs, pos_table[None])
